# Initial kernel scaffold; baseline (speedup 1.0000x reference)
import jax
import jax.numpy as jnp
from jax.experimental import pallas as pl


def kernel(x, edge_index, batch, W1_0, b1_0, W2_0, b2_0, W1_r, b1_r, W2_r, b2_r, bn_g, bn_b, ln_g, ln_b, Wf1, bf1, Wf2, bf2, lnf_g, lnf_b):
    raise NotImplementedError("write your pallas kernel here")


# speedup vs baseline: 2.0431x; 2.0431x over previous
import functools

import jax
import jax.numpy as jnp
from jax import lax
from jax.experimental import pallas as pl
from jax.experimental.pallas import tpu as pltpu
from jax.experimental.pallas import tpu_sc as plsc

N = 10000
E = 160000
F = 256
H = 512
O = 256
L = 4
G = 64

NCORE = 2
NSUB = 16
NW = NCORE * NSUB
CT = 80
NCHUNK = 4
NRANGE = NCHUNK * NW
K = 64

BN = 400
NB = N // BN


def _make_agg(D):
    mesh = plsc.VectorSubcoreMesh(core_axis_name="c", subcore_axis_name="s")

    @functools.partial(
        pl.kernel,
        out_type=jax.ShapeDtypeStruct((NRANGE * CT, D), jnp.float32),
        mesh=mesh,
        compiler_params=pltpu.CompilerParams(needs_layout_passes=False),
        scratch_types=[
            pltpu.VMEM((2, K), jnp.int32),
            pltpu.VMEM((2, K), jnp.int32),
            pltpu.VMEM((2, K, D), jnp.float32),
            pltpu.VMEM((NRANGE + 32,), jnp.int32),
            pltpu.VMEM((CT, D), jnp.float32),
            pltpu.SemaphoreType.DMA,
            pltpu.SemaphoreType.DMA,
        ],
    )
    def agg_kernel(src_hbm, dstm_hbm, off_hbm, h_hbm, out_hbm,
                   srcv, dstv, rows, offv, accum, gsem0, gsem1):
        core = lax.axis_index("c")
        sub = lax.axis_index("s")
        wid = core * NSUB + sub
        sems = (gsem0, gsem1)

        pltpu.sync_copy(off_hbm, offv)
        lane = lax.iota(jnp.int32, 16)

        def _process(rid):
            ev = offv[pl.ds(rid, 16)]
            e_lo = ev[0]
            e_hi = ev[1]
            a8 = (e_lo // 8) * 8
            nbat = (e_hi - a8 + K - 1) // K

            def start(j, b):
                e0 = a8 + j * K
                pltpu.sync_copy(src_hbm.at[pl.ds(e0, K)], srcv.at[b])
                pltpu.sync_copy(dstm_hbm.at[pl.ds(e0, K)], dstv.at[b])
                pltpu.async_copy(h_hbm.at[srcv.at[b]], rows.at[b], sems[b])

            def consume(j, b):
                pltpu.make_async_copy(
                    h_hbm.at[srcv.at[b]], rows.at[b], sems[b]).wait()
                e0 = a8 + j * K
                el = jnp.maximum(e_lo - e0, 0)
                eh = jnp.minimum(e_hi - e0, K)

                def sub16(s, _):
                    base = s * 16

                    @pl.when(base < eh)
                    def _():
                        for t in range(16):
                            e = base + t
                            sp = jnp.broadcast_to(e, (16,)).astype(jnp.int32)
                            dd = plsc.load_gather(dstv.at[b], [sp])
                            m = jnp.broadcast_to((e >= el) & (e < eh), (16,))
                            col = lane
                            for q in range(D // 16):
                                v = rows[b, e, pl.ds(q * 16, 16)]
                                plsc.addupdate_scatter(
                                    accum, [dd, col], v, mask=m)
                                col = col + 16
                    return 0
                lax.fori_loop(0, K // 16, sub16, 0)

            @pl.when(nbat > 0)
            def _():
                start(0, 0)

            pltpu.sync_copy(h_hbm.at[pl.ds(rid * CT, CT)], accum)

            def pair_body(p, _):
                for b in range(2):
                    j = 2 * p + b

                    @pl.when(j < nbat)
                    def _(j=j, b=b):
                        @pl.when(j + 1 < nbat)
                        def _():
                            start(j + 1, 1 - b)
                        consume(j, b)
                return 0
            lax.fori_loop(0, (nbat + 1) // 2, pair_body, 0)

            for q in range(CT // 16):
                pltpu.sync_copy(accum.at[pl.ds(q * 16, 16)],
                                out_hbm.at[pl.ds(rid * CT + q * 16, 16)])

        def chunk_body(c, _):
            rid = c * NW + wid
            pl.when(rid < (N // CT))(lambda: _process(rid))
            return 0
        lax.fori_loop(0, NCHUNK, chunk_body, 0)

    return agg_kernel


_agg_cache = {}


def _agg(D):
    if D not in _agg_cache:
        _agg_cache[D] = _make_agg(D)
    return _agg_cache[D]


def _mlp_body(z_ref, w1_ref, b1_ref, w2_ref, b2_ref,
              y_ref, s_ref, ss_ref):
    i = pl.program_id(0)
    z = z_ref[...]
    t = jnp.dot(z, w1_ref[...], preferred_element_type=jnp.float32)
    t = jnp.maximum(t + b1_ref[...], 0.0)
    y = jnp.dot(t, w2_ref[...], preferred_element_type=jnp.float32)
    y = y + b2_ref[...]
    y_ref[...] = y

    @pl.when(i == 0)
    def _():
        s_ref[...] = jnp.zeros_like(s_ref)
        ss_ref[...] = jnp.zeros_like(ss_ref)

    s_ref[...] += jnp.sum(y, axis=0, keepdims=True)
    ss_ref[...] += jnp.sum(y * y, axis=0, keepdims=True)


def _mlp(z_full, w1, b1, w2, b2):
    fin = z_full.shape[1]
    return pl.pallas_call(
        _mlp_body,
        grid=(NB,),
        in_specs=[
            pl.BlockSpec((BN, fin), lambda i: (i, 0)),
            pl.BlockSpec((fin, H), lambda i: (0, 0)),
            pl.BlockSpec((1, H), lambda i: (0, 0)),
            pl.BlockSpec((H, H), lambda i: (0, 0)),
            pl.BlockSpec((1, H), lambda i: (0, 0)),
        ],
        out_specs=[
            pl.BlockSpec((BN, H), lambda i: (i, 0)),
            pl.BlockSpec((1, H), lambda i: (0, 0)),
            pl.BlockSpec((1, H), lambda i: (0, 0)),
        ],
        out_shape=[
            jax.ShapeDtypeStruct((N, H), jnp.float32),
            jax.ShapeDtypeStruct((1, H), jnp.float32),
            jax.ShapeDtypeStruct((1, H), jnp.float32),
        ],
    )(z_full, w1, b1.reshape(1, H), w2, b2.reshape(1, H))


def _norm_body(y_ref, s_ref, ss_ref, bng_ref, bnb_ref, lng_ref, lnb_ref,
               o_ref):
    m = s_ref[...] / N
    v = ss_ref[...] / N - m * m
    y = y_ref[...]
    yn = (y - m) * lax.rsqrt(v + 1e-5) * bng_ref[...] + bnb_ref[...]
    yr = jnp.maximum(yn, 0.0)
    mu = jnp.mean(yr, axis=1, keepdims=True)
    var = jnp.mean(yr * yr, axis=1, keepdims=True) - mu * mu
    o_ref[...] = (yr - mu) * lax.rsqrt(var + 1e-5) * lng_ref[...] + lnb_ref[...]


def _norm(y, s, ss, bng, bnb, lng, lnb):
    return pl.pallas_call(
        _norm_body,
        grid=(NB,),
        in_specs=[
            pl.BlockSpec((BN, H), lambda i: (i, 0)),
            pl.BlockSpec((1, H), lambda i: (0, 0)),
            pl.BlockSpec((1, H), lambda i: (0, 0)),
            pl.BlockSpec((1, H), lambda i: (0, 0)),
            pl.BlockSpec((1, H), lambda i: (0, 0)),
            pl.BlockSpec((1, H), lambda i: (0, 0)),
            pl.BlockSpec((1, H), lambda i: (0, 0)),
        ],
        out_specs=pl.BlockSpec((BN, H), lambda i: (i, 0)),
        out_shape=jax.ShapeDtypeStruct((N, H), jnp.float32),
    )(y, s, ss, bng.reshape(1, H), bnb.reshape(1, H),
      lng.reshape(1, H), lnb.reshape(1, H))


def _pool_body(h_ref, batch_ref, wf1_ref, bf1_ref, wf2_ref, bf2_ref,
               lng_ref, lnb_ref, o_ref, p_acc):
    i = pl.program_id(0)

    @pl.when(i == 0)
    def _():
        p_acc[...] = jnp.zeros_like(p_acc)

    b = batch_ref[0, 0, :]
    seg = lax.broadcasted_iota(jnp.int32, (G, BN), 0)
    oh = (seg == b[None, :]).astype(jnp.float32)
    p_acc[...] += jnp.dot(oh, h_ref[...], preferred_element_type=jnp.float32)

    @pl.when(i == NB - 1)
    def _():
        p = p_acc[...]
        o1 = jnp.dot(p, wf1_ref[...], preferred_element_type=jnp.float32)
        o1 = jnp.maximum(o1 + bf1_ref[...], 0.0)
        o2 = jnp.dot(o1, wf2_ref[...], preferred_element_type=jnp.float32)
        o2 = jnp.maximum(o2 + bf2_ref[...], 0.0)
        mu = jnp.mean(o2, axis=1, keepdims=True)
        var = jnp.mean(o2 * o2, axis=1, keepdims=True) - mu * mu
        o_ref[...] = ((o2 - mu) * lax.rsqrt(var + 1e-5) * lng_ref[...]
                      + lnb_ref[...])


def _pool_head(h, batch2d, wf1, bf1, wf2, bf2, lng, lnb):
    return pl.pallas_call(
        _pool_body,
        grid=(NB,),
        in_specs=[
            pl.BlockSpec((BN, H), lambda i: (i, 0)),
            pl.BlockSpec((1, 1, BN), lambda i: (i, 0, 0)),
            pl.BlockSpec((H, H), lambda i: (0, 0)),
            pl.BlockSpec((1, H), lambda i: (0, 0)),
            pl.BlockSpec((H, O), lambda i: (0, 0)),
            pl.BlockSpec((1, O), lambda i: (0, 0)),
            pl.BlockSpec((1, O), lambda i: (0, 0)),
            pl.BlockSpec((1, O), lambda i: (0, 0)),
        ],
        out_specs=pl.BlockSpec((G, O), lambda i: (0, 0)),
        out_shape=jax.ShapeDtypeStruct((G, O), jnp.float32),
        scratch_shapes=[pltpu.VMEM((G, H), jnp.float32)],
    )(h, batch2d.reshape(NB, 1, BN), wf1, bf1.reshape(1, H), wf2,
      bf2.reshape(1, O), lng.reshape(1, O), lnb.reshape(1, O))


def kernel(x, edge_index, batch, W1_0, b1_0, W2_0, b2_0, W1_r, b1_r, W2_r,
           b2_r, bn_g, bn_b, ln_g, ln_b, Wf1, bf1, Wf2, bf2, lnf_g, lnf_b):
    src = edge_index[0]
    dst = edge_index[1]

    order = jnp.argsort(dst)
    src_s = src[order].astype(jnp.int32)
    dst_s = dst[order]
    off = jnp.searchsorted(
        dst_s,
        jnp.arange(NRANGE + 1, dtype=jnp.int32) * CT).astype(jnp.int32)
    off16 = jnp.concatenate(
        [off, jnp.full((31,), E, dtype=jnp.int32)])
    dstm_s = (dst_s % CT).astype(jnp.int32)
    pad = jnp.zeros((K,), jnp.int32)
    src_p = jnp.concatenate([src_s, pad])
    dstm_p = jnp.concatenate([dstm_s, pad])

    h = x
    for l in range(L):
        if l == 0:
            w1, b1, w2, b2 = W1_0, b1_0, W2_0, b2_0
            z_full = _agg(F)(src_p, dstm_p, off16, h)
        else:
            w1, b1, w2, b2 = W1_r[l - 1], b1_r[l - 1], W2_r[l - 1], b2_r[l - 1]
            z_full = _agg(H)(src_p, dstm_p, off16, h)
        y, s, ss = _mlp(z_full, w1, b1, w2, b2)
        h = _norm(y, s, ss, bn_g[l], bn_b[l], ln_g[l], ln_b[l])

    return _pool_head(h, batch.reshape(1, N).astype(jnp.int32),
                      Wf1, bf1, Wf2, bf2, lnf_g, lnf_b)
